# baseline (speedup 1.0000x reference)
import jax
import jax.numpy as jnp
from jax.experimental import pallas as pl
from jax.experimental.pallas import tpu as pltpu

_TM = 2048
_TH = _TM // 2


def _embed_kernel(a0, a1, we_ref, out_ref):
    off = (pl.program_id(0) % 2) * _TM
    w = we_ref[...].astype(jnp.bfloat16)
    out_ref[pl.ds(off, _TH), :] = jnp.dot(
        a0[...].astype(jnp.bfloat16), w, preferred_element_type=jnp.float32
    )
    out_ref[pl.ds(off + _TH, _TH), :] = jnp.dot(
        a1[...].astype(jnp.bfloat16), w, preferred_element_type=jnp.float32
    )


def kernel(adj, W_E):
    B, N, N2 = adj.shape
    D = W_E.shape[1]
    M = B * N
    adj2 = adj.reshape(M, N)

    out = pl.pallas_call(
        _embed_kernel,
        out_shape=jax.ShapeDtypeStruct((M, D), jnp.float32),
        grid=(M // _TM,),
        in_specs=[
            pl.BlockSpec((_TH, N), lambda i: (2 * i, 0)),
            pl.BlockSpec((_TH, N), lambda i: (2 * i + 1, 0)),
            pl.BlockSpec((N, D), lambda i: (0, 0)),
        ],
        out_specs=pl.BlockSpec((2 * _TM, D), lambda i: (i // 2, 0)),
        compiler_params=pltpu.CompilerParams(
            dimension_semantics=("parallel",),
        ),
        cost_estimate=pl.CostEstimate(
            flops=2 * M * N * D,
            transcendentals=0,
            bytes_accessed=adj.size * 4 + W_E.size * 4 + M * D * 4,
        ),
    )(adj2, adj2, W_E)

    return out.reshape(B, N, D)


# speedup vs baseline: 1.0282x; 1.0282x over previous
import jax
import jax.numpy as jnp
from jax.experimental import pallas as pl
from jax.experimental.pallas import tpu as pltpu

_TM = 2048
_TH = _TM // 2


def _embed_kernel(a0_ref, a1_ref, we_ref, out_ref):
    w = we_ref[...].astype(jnp.bfloat16)
    out_ref[:_TH, :] = jnp.dot(
        a0_ref[...].astype(jnp.bfloat16), w, preferred_element_type=jnp.float32
    )
    out_ref[_TH:, :] = jnp.dot(
        a1_ref[...].astype(jnp.bfloat16), w, preferred_element_type=jnp.float32
    )


def kernel(adj, W_E):
    B, N, N2 = adj.shape
    assert N2 == N
    D = W_E.shape[1]
    M = B * N
    assert M % _TM == 0

    adj2 = adj.reshape(M, N)

    out = pl.pallas_call(
        _embed_kernel,
        out_shape=jax.ShapeDtypeStruct((M, D), jnp.float32),
        grid=(M // _TM,),
        in_specs=[
            pl.BlockSpec((_TH, N), lambda i: (2 * i, 0)),
            pl.BlockSpec((_TH, N), lambda i: (2 * i + 1, 0)),
            pl.BlockSpec((N, D), lambda i: (0, 0)),
        ],
        out_specs=pl.BlockSpec((_TM, D), lambda i: (i, 0)),
        compiler_params=pltpu.CompilerParams(
            dimension_semantics=("parallel",),
        ),
        cost_estimate=pl.CostEstimate(
            flops=2 * M * N * D,
            transcendentals=0,
            bytes_accessed=adj.size * 4 + W_E.size * 4 + M * D * 4,
        ),
    )(adj2, adj2, W_E)

    return out.reshape(B, N, D)
